# Initial kernel scaffold; baseline (speedup 1.0000x reference)
import jax
import jax.numpy as jnp
from jax.experimental import pallas as pl


def kernel(seq, seq_len, edge_index, graph_ids, emb, Wih_f, Whh_f, b_f, Wih_b, Whh_b, b_b, W1, b1, W2, b2, W3, b3, Wr, br):
    raise NotImplementedError("write your pallas kernel here")


# speedup vs baseline: 6.3366x; 6.3366x over previous
import functools

import jax
import jax.numpy as jnp
from jax import lax
from jax.experimental import pallas as pl
from jax.experimental.pallas import tpu as pltpu
from jax.experimental.pallas import tpu_sc as plsc

N_NODES = 10000
N_EDGES = 320000
NPAD = 10240
ROWS_PER_TILE = NPAD // 16
EPAD = 327680
CHUNKS = EPAD // (16 * 128)
B = 128
T = 200
H = 64

_MESH = plsc.VectorSubcoreMesh(core_axis_name="c", subcore_axis_name="s")


def _zero_vec_ref(ref, n):
    def body(i, _):
        ref[pl.ds(i * 16, 16)] = jnp.zeros((16,), jnp.float32)
        return 0
    lax.fori_loop(0, n // 16, body, 0)


@functools.partial(
    pl.kernel,
    out_type=(jax.ShapeDtypeStruct((NPAD,), jnp.float32),
              jax.ShapeDtypeStruct((NPAD,), jnp.float32)),
    mesh=_MESH,
    compiler_params=pltpu.CompilerParams(use_tc_tiling_on_sc=False),
    scratch_types=[
        pltpu.VMEM((CHUNKS, 128), jnp.int32),
        pltpu.VMEM((CHUNKS, 128), jnp.int32),
        pltpu.VMEM((128,), jnp.float32),
        pltpu.VMEM((128,), jnp.float32),
        pltpu.VMEM((ROWS_PER_TILE,), jnp.float32),
        pltpu.VMEM_SHARED((NPAD,), jnp.float32),
        pltpu.VMEM_SHARED((NPAD,), jnp.float32),
    ],
)
def _deg_s1_kernel(srci, dsti, deg_o, s1_o,
                   srcv, dstv, valb, onesb, zb, deg_sh, s1_sh):
    c = lax.axis_index("c")

    @pl.when(c == 0)
    def _core0():
        s = lax.axis_index("s")
        _zero_vec_ref(zb, ROWS_PER_TILE)

        def ones16(i, _):
            onesb[pl.ds(i * 16, 16)] = jnp.ones((16,), jnp.float32)
            return 0
        lax.fori_loop(0, 8, ones16, 0)

        pltpu.sync_copy(zb, deg_sh.at[pl.ds(s * ROWS_PER_TILE, ROWS_PER_TILE)])
        pltpu.sync_copy(zb, s1_sh.at[pl.ds(s * ROWS_PER_TILE, ROWS_PER_TILE)])
        pltpu.sync_copy(srci.at[s], srcv)
        pltpu.sync_copy(dsti.at[s], dstv)
        plsc.subcore_barrier()

        def ph1(ck, _):
            pltpu.sync_copy(onesb, deg_sh.at[dstv.at[ck]], add=True)
            return 0
        lax.fori_loop(0, CHUNKS, ph1, 0)
        plsc.subcore_barrier()

        sl = pl.ds(s * ROWS_PER_TILE, ROWS_PER_TILE)
        pltpu.sync_copy(deg_sh.at[sl], deg_o.at[sl])
        plsc.subcore_barrier()

        def ph2(ck, _):
            pltpu.sync_copy(deg_o.at[srcv.at[ck]], valb)
            pltpu.sync_copy(valb, s1_sh.at[dstv.at[ck]], add=True)
            return 0
        lax.fori_loop(0, CHUNKS, ph2, 0)
        plsc.subcore_barrier()
        pltpu.sync_copy(s1_sh.at[sl], s1_o.at[sl])


def _make_agg(dh):
    @functools.partial(
        pl.kernel,
        out_type=(jax.ShapeDtypeStruct((NPAD, dh), jnp.float32),
                  jax.ShapeDtypeStruct((NPAD, dh), jnp.float32)),
        mesh=_MESH,
        compiler_params=pltpu.CompilerParams(use_tc_tiling_on_sc=False),
        scratch_types=[
            pltpu.VMEM((CHUNKS, 128), jnp.int32),
            pltpu.VMEM((CHUNKS, 128), jnp.int32),
            pltpu.VMEM((128, dh), jnp.float32),
            pltpu.VMEM_SHARED((NPAD, dh), jnp.float32),
        ],
    )
    def agg(ha, hb, srci, dsti, sa_o, sb_o, srcv, dstv, gbuf, s_sh):
        c = lax.axis_index("c")
        s = lax.axis_index("s")

        def zr(i, _):
            def zc(j, _):
                gbuf[i, pl.ds(j * 16, 16)] = jnp.zeros((16,), jnp.float32)
                return 0
            lax.fori_loop(0, dh // 16, zc, 0)
            return 0
        lax.fori_loop(0, 128, zr, 0)
        for r in range(ROWS_PER_TILE // 128):
            pltpu.sync_copy(gbuf, s_sh.at[pl.ds(s * ROWS_PER_TILE + r * 128, 128)])
        pltpu.sync_copy(srci.at[s], srcv)
        pltpu.sync_copy(dsti.at[s], dstv)
        plsc.subcore_barrier()

        def run(h_hbm):
            def ch(ck, _):
                pltpu.sync_copy(h_hbm.at[srcv.at[ck]], gbuf)
                pltpu.sync_copy(gbuf, s_sh.at[dstv.at[ck]], add=True)
                return 0
            lax.fori_loop(0, CHUNKS, ch, 0)

        @pl.when(c == 0)
        def _():
            run(ha)

        @pl.when(c == 1)
        def _():
            run(hb)

        plsc.subcore_barrier()
        sl = pl.ds(s * ROWS_PER_TILE, ROWS_PER_TILE)

        @pl.when(c == 0)
        def _():
            pltpu.sync_copy(s_sh.at[sl], sa_o.at[sl])

        @pl.when(c == 1)
        def _():
            pltpu.sync_copy(s_sh.at[sl], sb_o.at[sl])

    return agg


_agg64 = _make_agg(64)


def _layer1_body(deg_ref, s1_ref, w1_ref, b1_ref, out_ref):
    d = deg_ref[...]
    a = jnp.where(d > 0, s1_ref[...] / jnp.maximum(d, 1.0), d)
    out_ref[...] = jnp.maximum(a * w1_ref[...] + b1_ref[...], 0.0)


_RB = 1280
_NRB = NPAD // _RB


def _layer2_body(s2a_ref, s2b_ref, deg_ref, h1_ref, w2t_ref, b2_ref, out_ref):
    d = deg_ref[...]
    s2 = jnp.concatenate([s2a_ref[...], s2b_ref[...]], axis=1)
    a = jnp.where(d > 0, s2 / jnp.maximum(d, 1.0), h1_ref[...])
    out_ref[...] = jnp.maximum(
        jnp.dot(a, w2t_ref[...], preferred_element_type=jnp.float32,
                precision=lax.Precision.HIGHEST)
        + b2_ref[...], 0.0)


def _layer3_body(s3a_ref, s3b_ref, s3c_ref, s3d_ref, deg_ref, h2_ref,
                 w3t_ref, b3_ref, gid_ref,
                 co_ref, wra_ref, wrb_ref, br_ref, out_ref,
                 pool_ref, cnt_ref):
    t = pl.program_id(0)
    prec = lax.Precision.HIGHEST

    @pl.when(t == 0)
    def _init():
        pool_ref[...] = jnp.zeros((B, B), jnp.float32)
        cnt_ref[...] = jnp.zeros((B, 1), jnp.float32)

    d = deg_ref[...]
    s3 = jnp.concatenate([s3a_ref[...], s3b_ref[...], s3c_ref[...],
                          s3d_ref[...]], axis=1)
    a = jnp.where(d > 0, s3 / jnp.maximum(d, 1.0), h2_ref[...])
    h3 = jnp.maximum(
        jnp.dot(a, w3t_ref[...], preferred_element_type=jnp.float32,
                precision=prec)
        + b3_ref[...], 0.0)
    onehot = (gid_ref[...] == lax.broadcasted_iota(jnp.int32, (_RB, B), 1)
              ).astype(jnp.float32)
    dims = (((0,), (0,)), ((), ()))
    pool_ref[...] += lax.dot_general(onehot, h3, dims,
                                     preferred_element_type=jnp.float32,
                                     precision=prec)
    cnt_ref[...] += lax.dot_general(onehot, jnp.ones((_RB, 1), jnp.float32),
                                    dims, preferred_element_type=jnp.float32,
                                    precision=prec)

    @pl.when(t == _NRB - 1)
    def _fin():
        gmean = pool_ref[...] / jnp.maximum(cnt_ref[...], 1.0)
        out_ref[...] = (
            jnp.dot(co_ref[...], wra_ref[...],
                    preferred_element_type=jnp.float32, precision=prec)
            + jnp.dot(gmean, wrb_ref[...],
                      preferred_element_type=jnp.float32, precision=prec)
            + br_ref[...])


def _lstm_body(seq_ref, embp_ref, wft_ref, wbt_ref, bfull_ref,
               whh_ref, lo_ref, hi_ref, out_ref, tab_ref, h_ref, c_ref):
    t = pl.program_id(0)

    @pl.when(t == 0)
    def _init():
        tf = jnp.dot(embp_ref[...], wft_ref[...],
                     preferred_element_type=jnp.float32)
        tb = jnp.dot(embp_ref[...], wbt_ref[...],
                     preferred_element_type=jnp.float32)
        tab_ref[...] = jnp.concatenate([tf, tb], axis=0)
        h_ref[...] = jnp.zeros((2 * B, H), jnp.float32)
        c_ref[...] = jnp.zeros((2 * B, H), jnp.float32)

    sv = seq_ref[0, :, :]
    isb = lax.broadcasted_iota(jnp.int32, (1, 2 * B), 1) >= B
    tgt = sv + jnp.where(isb, 32, 0)
    oh_t = (lax.broadcasted_iota(jnp.int32, (64, 2 * B), 0) == tgt
            ).astype(jnp.float32)
    rowi = lax.broadcasted_iota(jnp.int32, (2 * B, 1), 0)
    mf = (rowi < B).astype(jnp.float32)
    h = h_ref[...]
    c = c_ref[...]
    hbig = jnp.concatenate([h * mf, h * (1.0 - mf)], axis=1)
    dims = (((0,), (0,)), ((), ()))
    gates = (lax.dot_general(oh_t, tab_ref[...], dims,
                             preferred_element_type=jnp.float32,
                             precision=lax.Precision.HIGHEST)
             + jnp.dot(hbig, whh_ref[...],
                       preferred_element_type=jnp.float32)) + bfull_ref[...]
    i_g = gates[:, 0:H]
    f_g = gates[:, H:2 * H]
    g_g = gates[:, 2 * H:3 * H]
    o_g = gates[:, 3 * H:4 * H]
    c_new = jax.nn.sigmoid(f_g) * c + jax.nn.sigmoid(i_g) * jnp.tanh(g_g)
    h_new = jax.nn.sigmoid(o_g) * jnp.tanh(c_new)
    m = jnp.logical_and(t >= lo_ref[...], t < hi_ref[...])
    h_ref[...] = jnp.where(m, h_new, h)
    c_ref[...] = jnp.where(m, c_new, c)

    @pl.when(t == T - 1)
    def _fin():
        out_ref[...] = h_ref[...]


def _run_lstm(seq2r, embp, wft, wbt, bfull, whh_cat, lo, hi):
    return pl.pallas_call(
        _lstm_body,
        grid=(T,),
        in_specs=[
            pl.BlockSpec((1, 1, 2 * B), lambda t: (t, 0, 0)),
            pl.BlockSpec((32, 16), lambda t: (0, 0)),
            pl.BlockSpec((16, 4 * H), lambda t: (0, 0)),
            pl.BlockSpec((16, 4 * H), lambda t: (0, 0)),
            pl.BlockSpec((2 * B, 4 * H), lambda t: (0, 0)),
            pl.BlockSpec((2 * H, 4 * H), lambda t: (0, 0)),
            pl.BlockSpec((2 * B, 1), lambda t: (0, 0)),
            pl.BlockSpec((2 * B, 1), lambda t: (0, 0)),
        ],
        out_specs=pl.BlockSpec((2 * B, H), lambda t: (0, 0)),
        out_shape=jax.ShapeDtypeStruct((2 * B, H), jnp.float32),
        scratch_shapes=[
            pltpu.VMEM((64, 4 * H), jnp.float32),
            pltpu.VMEM((2 * B, H), jnp.float32),
            pltpu.VMEM((2 * B, H), jnp.float32),
        ],
    )(seq2r, embp, wft, wbt, bfull, whh_cat, lo, hi)


def kernel(seq, seq_len, edge_index, graph_ids, emb, Wih_f, Whh_f, b_f,
           Wih_b, Whh_b, b_b, W1, b1, W2, b2, W3, b3, Wr, br):
    f32 = jnp.float32

    npadd = EPAD - N_EDGES
    dummy = (N_NODES + (jnp.arange(npadd, dtype=jnp.int32) % (NPAD - N_NODES))
             ).astype(jnp.int32)
    src = jnp.concatenate([edge_index[0].astype(jnp.int32), dummy])
    dst = jnp.concatenate([edge_index[1].astype(jnp.int32), dummy])
    srci = src.reshape(16, CHUNKS, 128)
    dsti = dst.reshape(16, CHUNKS, 128)

    deg, s1 = _deg_s1_kernel(srci, dsti)
    degc = deg.reshape(NPAD, 1)

    h1 = pl.pallas_call(
        _layer1_body,
        out_shape=jax.ShapeDtypeStruct((NPAD, 128), f32),
    )(degc, s1.reshape(NPAD, 1), W1.astype(f32).reshape(1, 128),
      b1.astype(f32).reshape(1, 128))

    s2a, s2b = _agg64(h1[:, :64], h1[:, 64:], srci, dsti)

    rb = pl.BlockSpec((_RB, 64), lambda t: (t, 0))
    rb1 = pl.BlockSpec((_RB, 1), lambda t: (t, 0))
    rb128 = pl.BlockSpec((_RB, 128), lambda t: (t, 0))
    rb256 = pl.BlockSpec((_RB, 256), lambda t: (t, 0))
    h2 = pl.pallas_call(
        _layer2_body,
        grid=(_NRB,),
        in_specs=[rb, rb, rb1, rb128,
                  pl.BlockSpec((128, 256), lambda t: (0, 0)),
                  pl.BlockSpec((1, 256), lambda t: (0, 0))],
        out_specs=rb256,
        out_shape=jax.ShapeDtypeStruct((NPAD, 256), f32),
    )(s2a, s2b, degc, h1, W2.astype(f32).T, b2.astype(f32).reshape(1, 256))

    s3a, s3b = _agg64(h2[:, 0:64], h2[:, 64:128], srci, dsti)
    s3c, s3d = _agg64(h2[:, 128:192], h2[:, 192:256], srci, dsti)

    a = jnp.swapaxes(seq, 0, 1).astype(jnp.int32)
    seq2r = jnp.concatenate([a, a[::-1]], axis=1).reshape(T, 1, 2 * B)
    embp = jnp.zeros((32, 16), f32).at[0:21, 0:10].set(emb.astype(f32))
    wft = jnp.zeros((16, 4 * H), f32).at[0:10].set(Wih_f.astype(f32).T)
    wbt = jnp.zeros((16, 4 * H), f32).at[0:10].set(Wih_b.astype(f32).T)
    sl = seq_len.astype(jnp.int32)
    lo = jnp.concatenate([jnp.zeros((B,), jnp.int32), T - sl]).reshape(2 * B, 1)
    hi = jnp.concatenate([sl, jnp.full((B,), T, jnp.int32)]).reshape(2 * B, 1)
    whh_cat = jnp.concatenate(
        [Whh_f.astype(f32).T, Whh_b.astype(f32).T], axis=0)
    bfull = jnp.concatenate(
        [jnp.broadcast_to(b_f.astype(f32), (B, 4 * H)),
         jnp.broadcast_to(b_b.astype(f32), (B, 4 * H))], axis=0)
    hout = _run_lstm(seq2r, embp, wft, wbt, bfull, whh_cat, lo, hi)
    concat_o = jnp.concatenate([hout[:B], hout[B:]], axis=1)

    gidp = jnp.concatenate(
        [graph_ids.astype(jnp.int32),
         jnp.full((NPAD - N_NODES,), 1 << 29, jnp.int32)]).reshape(NPAD, 1)
    out = pl.pallas_call(
        _layer3_body,
        grid=(_NRB,),
        in_specs=[rb, rb, rb, rb, rb1, rb256,
                  pl.BlockSpec((256, 128), lambda t: (0, 0)),
                  pl.BlockSpec((1, 128), lambda t: (0, 0)),
                  pl.BlockSpec((_RB, 1), lambda t: (t, 0)),
                  pl.BlockSpec((B, B), lambda t: (0, 0)),
                  pl.BlockSpec((B, 1), lambda t: (0, 0)),
                  pl.BlockSpec((B, 1), lambda t: (0, 0)),
                  pl.BlockSpec((1, 1), lambda t: (0, 0))],
        out_specs=pl.BlockSpec((B, 1), lambda t: (0, 0)),
        out_shape=jax.ShapeDtypeStruct((B, 1), f32),
        scratch_shapes=[pltpu.VMEM((B, B), f32), pltpu.VMEM((B, 1), f32)],
    )(s3a, s3b, s3c, s3d, degc, h2,
      W3.astype(f32).T, b3.astype(f32).reshape(1, 128), gidp,
      concat_o, Wr.astype(f32)[:, :128].T, Wr.astype(f32)[:, 128:].T,
      br.astype(f32).reshape(1, 1))
    return out


# speedup vs baseline: 11.3399x; 1.7896x over previous
import functools

import jax
import jax.numpy as jnp
from jax import lax
from jax.experimental import pallas as pl
from jax.experimental.pallas import tpu as pltpu
from jax.experimental.pallas import tpu_sc as plsc

N_NODES = 10000
N_EDGES = 320000
NPAD = 10240
ROWS_PER_TILE = NPAD // 16
EPAD = 327680
CHUNKS = EPAD // (16 * 128)
B = 128
T = 200
H = 64

_MESH = plsc.VectorSubcoreMesh(core_axis_name="c", subcore_axis_name="s")


def _zero_vec_ref(ref, n):
    def body(i, _):
        ref[pl.ds(i * 16, 16)] = jnp.zeros((16,), jnp.float32)
        return 0
    lax.fori_loop(0, n // 16, body, 0)


@functools.partial(
    pl.kernel,
    out_type=(jax.ShapeDtypeStruct((NPAD,), jnp.float32),
              jax.ShapeDtypeStruct((NPAD,), jnp.float32)),
    mesh=_MESH,
    compiler_params=pltpu.CompilerParams(use_tc_tiling_on_sc=False),
    scratch_types=[
        pltpu.VMEM((CHUNKS, 128), jnp.int32),
        pltpu.VMEM((CHUNKS, 128), jnp.int32),
        pltpu.VMEM((4, 128), jnp.float32),
        pltpu.VMEM((128,), jnp.float32),
        pltpu.VMEM((ROWS_PER_TILE,), jnp.float32),
        pltpu.VMEM_SHARED((NPAD,), jnp.float32),
        pltpu.VMEM_SHARED((NPAD,), jnp.float32),
        [pltpu.SemaphoreType.DMA] * 4,
        [pltpu.SemaphoreType.DMA] * 4,
    ],
)
def _deg_s1_kernel(srci, dsti, deg_o, s1_o,
                   srcv, dstv, valb, onesb, zb, deg_sh, s1_sh, gsem, ssem):
    c = lax.axis_index("c")

    @pl.when(c == 0)
    def _core0():
        s = lax.axis_index("s")
        _zero_vec_ref(zb, ROWS_PER_TILE)

        def ones16(i, _):
            onesb[pl.ds(i * 16, 16)] = jnp.ones((16,), jnp.float32)
            return 0
        lax.fori_loop(0, 8, ones16, 0)

        pltpu.sync_copy(zb, deg_sh.at[pl.ds(s * ROWS_PER_TILE, ROWS_PER_TILE)])
        pltpu.sync_copy(zb, s1_sh.at[pl.ds(s * ROWS_PER_TILE, ROWS_PER_TILE)])
        pltpu.sync_copy(srci.at[s], srcv)
        pltpu.sync_copy(dsti.at[s], dstv)
        plsc.subcore_barrier()

        def ph1(ck, _):
            pltpu.async_copy(onesb, deg_sh.at[dstv.at[ck]], ssem[0], add=True)
            return 0
        lax.fori_loop(0, CHUNKS, ph1, 0)

        def ph1w(ck, _):
            pltpu.make_async_copy(onesb, deg_sh.at[dstv.at[ck]],
                                  ssem[0]).wait()
            return 0
        lax.fori_loop(0, CHUNKS, ph1w, 0)
        plsc.subcore_barrier()

        sl = pl.ds(s * ROWS_PER_TILE, ROWS_PER_TILE)
        pltpu.sync_copy(deg_sh.at[sl], deg_o.at[sl])
        plsc.subcore_barrier()

        nb = 4
        rounds = CHUNKS // nb
        for b in range(nb):
            pltpu.async_copy(deg_o.at[srcv.at[b]], valb.at[b], gsem[b])

        def ph2(r, _):
            for b in range(nb):
                ck = r * nb + b
                pltpu.make_async_copy(
                    deg_o.at[srcv.at[ck]], valb.at[b], gsem[b]).wait()
                pltpu.async_copy(valb.at[b], s1_sh.at[dstv.at[ck]],
                                 ssem[b], add=True)

            @pl.when(r < rounds - 1)
            def _next():
                for b in range(nb):
                    ck = r * nb + b
                    pltpu.make_async_copy(
                        valb.at[b], s1_sh.at[dstv.at[ck]], ssem[b]).wait()
                    pltpu.async_copy(deg_o.at[srcv.at[ck + nb]],
                                     valb.at[b], gsem[b])
            return 0
        lax.fori_loop(0, rounds, ph2, 0)
        for b in range(nb):
            ck = CHUNKS - nb + b
            pltpu.make_async_copy(
                valb.at[b], s1_sh.at[dstv.at[ck]], ssem[b]).wait()
        plsc.subcore_barrier()
        pltpu.sync_copy(s1_sh.at[sl], s1_o.at[sl])


def _make_agg(dh):
    @functools.partial(
        pl.kernel,
        out_type=(jax.ShapeDtypeStruct((NPAD, dh), jnp.float32),
                  jax.ShapeDtypeStruct((NPAD, dh), jnp.float32)),
        mesh=_MESH,
        compiler_params=pltpu.CompilerParams(use_tc_tiling_on_sc=False),
        scratch_types=[
            pltpu.VMEM((CHUNKS, 128), jnp.int32),
            pltpu.VMEM((CHUNKS, 128), jnp.int32),
            pltpu.VMEM((4, 128, dh), jnp.float32),
            pltpu.VMEM_SHARED((NPAD, dh), jnp.float32),
            [pltpu.SemaphoreType.DMA] * 4,
            [pltpu.SemaphoreType.DMA] * 4,
        ],
    )
    def agg(ha, hb, srci, dsti, sa_o, sb_o, srcv, dstv, gbuf, s_sh,
            gsem, ssem):
        c = lax.axis_index("c")
        s = lax.axis_index("s")

        def zr(i, _):
            def zc(j, _):
                gbuf[0, i, pl.ds(j * 16, 16)] = jnp.zeros((16,), jnp.float32)
                return 0
            lax.fori_loop(0, dh // 16, zc, 0)
            return 0
        lax.fori_loop(0, 128, zr, 0)
        for r in range(ROWS_PER_TILE // 128):
            pltpu.sync_copy(gbuf.at[0],
                            s_sh.at[pl.ds(s * ROWS_PER_TILE + r * 128, 128)])
        pltpu.sync_copy(srci.at[s], srcv)
        pltpu.sync_copy(dsti.at[s], dstv)
        plsc.subcore_barrier()

        def run(h_hbm):
            nb = 4
            rounds = CHUNKS // nb
            for b in range(nb):
                pltpu.async_copy(h_hbm.at[srcv.at[b]], gbuf.at[b], gsem[b])

            def rnd(r, _):
                for b in range(nb):
                    ck = r * nb + b
                    pltpu.make_async_copy(
                        h_hbm.at[srcv.at[ck]], gbuf.at[b], gsem[b]).wait()
                    pltpu.async_copy(gbuf.at[b], s_sh.at[dstv.at[ck]],
                                     ssem[b], add=True)

                @pl.when(r < rounds - 1)
                def _next():
                    for b in range(nb):
                        ck = r * nb + b
                        pltpu.make_async_copy(
                            gbuf.at[b], s_sh.at[dstv.at[ck]], ssem[b]).wait()
                        pltpu.async_copy(h_hbm.at[srcv.at[ck + nb]],
                                         gbuf.at[b], gsem[b])
                return 0
            lax.fori_loop(0, rounds, rnd, 0)
            for b in range(nb):
                ck = CHUNKS - nb + b
                pltpu.make_async_copy(
                    gbuf.at[b], s_sh.at[dstv.at[ck]], ssem[b]).wait()

        @pl.when(c == 0)
        def _():
            run(ha)

        @pl.when(c == 1)
        def _():
            run(hb)

        plsc.subcore_barrier()
        sl = pl.ds(s * ROWS_PER_TILE, ROWS_PER_TILE)

        @pl.when(c == 0)
        def _():
            pltpu.sync_copy(s_sh.at[sl], sa_o.at[sl])

        @pl.when(c == 1)
        def _():
            pltpu.sync_copy(s_sh.at[sl], sb_o.at[sl])

    return agg


_agg64 = _make_agg(64)


def _layer1_body(deg_ref, s1_ref, w1_ref, b1_ref, out_ref):
    d = deg_ref[...]
    a = jnp.where(d > 0, s1_ref[...] / jnp.maximum(d, 1.0), d)
    out_ref[...] = jnp.maximum(a * w1_ref[...] + b1_ref[...], 0.0)


_RB = 1280
_NRB = NPAD // _RB


def _layer2_body(s2a_ref, s2b_ref, deg_ref, h1_ref, w2t_ref, b2_ref, out_ref):
    d = deg_ref[...]
    s2 = jnp.concatenate([s2a_ref[...], s2b_ref[...]], axis=1)
    a = jnp.where(d > 0, s2 / jnp.maximum(d, 1.0), h1_ref[...])
    out_ref[...] = jnp.maximum(
        jnp.dot(a, w2t_ref[...], preferred_element_type=jnp.float32,
                precision=lax.Precision.HIGHEST)
        + b2_ref[...], 0.0)


def _layer3_body(s3a_ref, s3b_ref, s3c_ref, s3d_ref, deg_ref, h2_ref,
                 w3t_ref, b3_ref, gid_ref,
                 co_ref, wra_ref, wrb_ref, br_ref, out_ref,
                 pool_ref, cnt_ref):
    t = pl.program_id(0)
    prec = lax.Precision.HIGHEST

    @pl.when(t == 0)
    def _init():
        pool_ref[...] = jnp.zeros((B, B), jnp.float32)
        cnt_ref[...] = jnp.zeros((B, 1), jnp.float32)

    d = deg_ref[...]
    s3 = jnp.concatenate([s3a_ref[...], s3b_ref[...], s3c_ref[...],
                          s3d_ref[...]], axis=1)
    a = jnp.where(d > 0, s3 / jnp.maximum(d, 1.0), h2_ref[...])
    h3 = jnp.maximum(
        jnp.dot(a, w3t_ref[...], preferred_element_type=jnp.float32,
                precision=prec)
        + b3_ref[...], 0.0)
    onehot = (gid_ref[...] == lax.broadcasted_iota(jnp.int32, (_RB, B), 1)
              ).astype(jnp.float32)
    dims = (((0,), (0,)), ((), ()))
    pool_ref[...] += lax.dot_general(onehot, h3, dims,
                                     preferred_element_type=jnp.float32,
                                     precision=prec)
    cnt_ref[...] += lax.dot_general(onehot, jnp.ones((_RB, 1), jnp.float32),
                                    dims, preferred_element_type=jnp.float32,
                                    precision=prec)

    @pl.when(t == _NRB - 1)
    def _fin():
        gmean = pool_ref[...] / jnp.maximum(cnt_ref[...], 1.0)
        out_ref[...] = (
            jnp.dot(co_ref[...], wra_ref[...],
                    preferred_element_type=jnp.float32, precision=prec)
            + jnp.dot(gmean, wrb_ref[...],
                      preferred_element_type=jnp.float32, precision=prec)
            + br_ref[...])


def _lstm_body(seq_ref, embp_ref, wft_ref, wbt_ref, bfull_ref,
               whh_ref, lo_ref, hi_ref, out_ref, tab_ref, h_ref, c_ref):
    t = pl.program_id(0)

    @pl.when(t == 0)
    def _init():
        tf = jnp.dot(embp_ref[...], wft_ref[...],
                     preferred_element_type=jnp.float32)
        tb = jnp.dot(embp_ref[...], wbt_ref[...],
                     preferred_element_type=jnp.float32)
        tab_ref[...] = jnp.concatenate([tf, tb], axis=0)
        h_ref[...] = jnp.zeros((2 * B, H), jnp.float32)
        c_ref[...] = jnp.zeros((2 * B, H), jnp.float32)

    sv = seq_ref[0, :, :]
    isb = lax.broadcasted_iota(jnp.int32, (1, 2 * B), 1) >= B
    tgt = sv + jnp.where(isb, 32, 0)
    oh_t = (lax.broadcasted_iota(jnp.int32, (64, 2 * B), 0) == tgt
            ).astype(jnp.float32)
    rowi = lax.broadcasted_iota(jnp.int32, (2 * B, 1), 0)
    mf = (rowi < B).astype(jnp.float32)
    h = h_ref[...]
    c = c_ref[...]
    hbig = jnp.concatenate([h * mf, h * (1.0 - mf)], axis=1)
    dims = (((0,), (0,)), ((), ()))
    gates = (lax.dot_general(oh_t, tab_ref[...], dims,
                             preferred_element_type=jnp.float32,
                             precision=lax.Precision.HIGHEST)
             + jnp.dot(hbig, whh_ref[...],
                       preferred_element_type=jnp.float32)) + bfull_ref[...]
    i_g = gates[:, 0:H]
    f_g = gates[:, H:2 * H]
    g_g = gates[:, 2 * H:3 * H]
    o_g = gates[:, 3 * H:4 * H]
    c_new = jax.nn.sigmoid(f_g) * c + jax.nn.sigmoid(i_g) * jnp.tanh(g_g)
    h_new = jax.nn.sigmoid(o_g) * jnp.tanh(c_new)
    m = jnp.logical_and(t >= lo_ref[...], t < hi_ref[...])
    h_ref[...] = jnp.where(m, h_new, h)
    c_ref[...] = jnp.where(m, c_new, c)

    @pl.when(t == T - 1)
    def _fin():
        out_ref[...] = h_ref[...]


def _run_lstm(seq2r, embp, wft, wbt, bfull, whh_cat, lo, hi):
    return pl.pallas_call(
        _lstm_body,
        grid=(T,),
        in_specs=[
            pl.BlockSpec((1, 1, 2 * B), lambda t: (t, 0, 0)),
            pl.BlockSpec((32, 16), lambda t: (0, 0)),
            pl.BlockSpec((16, 4 * H), lambda t: (0, 0)),
            pl.BlockSpec((16, 4 * H), lambda t: (0, 0)),
            pl.BlockSpec((2 * B, 4 * H), lambda t: (0, 0)),
            pl.BlockSpec((2 * H, 4 * H), lambda t: (0, 0)),
            pl.BlockSpec((2 * B, 1), lambda t: (0, 0)),
            pl.BlockSpec((2 * B, 1), lambda t: (0, 0)),
        ],
        out_specs=pl.BlockSpec((2 * B, H), lambda t: (0, 0)),
        out_shape=jax.ShapeDtypeStruct((2 * B, H), jnp.float32),
        scratch_shapes=[
            pltpu.VMEM((64, 4 * H), jnp.float32),
            pltpu.VMEM((2 * B, H), jnp.float32),
            pltpu.VMEM((2 * B, H), jnp.float32),
        ],
    )(seq2r, embp, wft, wbt, bfull, whh_cat, lo, hi)


def kernel(seq, seq_len, edge_index, graph_ids, emb, Wih_f, Whh_f, b_f,
           Wih_b, Whh_b, b_b, W1, b1, W2, b2, W3, b3, Wr, br):
    f32 = jnp.float32

    npadd = EPAD - N_EDGES
    dummy = (N_NODES + (jnp.arange(npadd, dtype=jnp.int32) % (NPAD - N_NODES))
             ).astype(jnp.int32)
    src = jnp.concatenate([edge_index[0].astype(jnp.int32), dummy])
    dst = jnp.concatenate([edge_index[1].astype(jnp.int32), dummy])
    srci = src.reshape(16, CHUNKS, 128)
    dsti = dst.reshape(16, CHUNKS, 128)

    deg, s1 = _deg_s1_kernel(srci, dsti)
    degc = deg.reshape(NPAD, 1)

    h1 = pl.pallas_call(
        _layer1_body,
        out_shape=jax.ShapeDtypeStruct((NPAD, 128), f32),
    )(degc, s1.reshape(NPAD, 1), W1.astype(f32).reshape(1, 128),
      b1.astype(f32).reshape(1, 128))

    s2a, s2b = _agg64(h1[:, :64], h1[:, 64:], srci, dsti)

    rb = pl.BlockSpec((_RB, 64), lambda t: (t, 0))
    rb1 = pl.BlockSpec((_RB, 1), lambda t: (t, 0))
    rb128 = pl.BlockSpec((_RB, 128), lambda t: (t, 0))
    rb256 = pl.BlockSpec((_RB, 256), lambda t: (t, 0))
    h2 = pl.pallas_call(
        _layer2_body,
        grid=(_NRB,),
        in_specs=[rb, rb, rb1, rb128,
                  pl.BlockSpec((128, 256), lambda t: (0, 0)),
                  pl.BlockSpec((1, 256), lambda t: (0, 0))],
        out_specs=rb256,
        out_shape=jax.ShapeDtypeStruct((NPAD, 256), f32),
    )(s2a, s2b, degc, h1, W2.astype(f32).T, b2.astype(f32).reshape(1, 256))

    s3a, s3b = _agg64(h2[:, 0:64], h2[:, 64:128], srci, dsti)
    s3c, s3d = _agg64(h2[:, 128:192], h2[:, 192:256], srci, dsti)

    a = jnp.swapaxes(seq, 0, 1).astype(jnp.int32)
    seq2r = jnp.concatenate([a, a[::-1]], axis=1).reshape(T, 1, 2 * B)
    embp = jnp.zeros((32, 16), f32).at[0:21, 0:10].set(emb.astype(f32))
    wft = jnp.zeros((16, 4 * H), f32).at[0:10].set(Wih_f.astype(f32).T)
    wbt = jnp.zeros((16, 4 * H), f32).at[0:10].set(Wih_b.astype(f32).T)
    sl = seq_len.astype(jnp.int32)
    lo = jnp.concatenate([jnp.zeros((B,), jnp.int32), T - sl]).reshape(2 * B, 1)
    hi = jnp.concatenate([sl, jnp.full((B,), T, jnp.int32)]).reshape(2 * B, 1)
    whh_cat = jnp.concatenate(
        [Whh_f.astype(f32).T, Whh_b.astype(f32).T], axis=0)
    bfull = jnp.concatenate(
        [jnp.broadcast_to(b_f.astype(f32), (B, 4 * H)),
         jnp.broadcast_to(b_b.astype(f32), (B, 4 * H))], axis=0)
    hout = _run_lstm(seq2r, embp, wft, wbt, bfull, whh_cat, lo, hi)
    concat_o = jnp.concatenate([hout[:B], hout[B:]], axis=1)

    gidp = jnp.concatenate(
        [graph_ids.astype(jnp.int32),
         jnp.full((NPAD - N_NODES,), 1 << 29, jnp.int32)]).reshape(NPAD, 1)
    out = pl.pallas_call(
        _layer3_body,
        grid=(_NRB,),
        in_specs=[rb, rb, rb, rb, rb1, rb256,
                  pl.BlockSpec((256, 128), lambda t: (0, 0)),
                  pl.BlockSpec((1, 128), lambda t: (0, 0)),
                  pl.BlockSpec((_RB, 1), lambda t: (t, 0)),
                  pl.BlockSpec((B, B), lambda t: (0, 0)),
                  pl.BlockSpec((B, 1), lambda t: (0, 0)),
                  pl.BlockSpec((B, 1), lambda t: (0, 0)),
                  pl.BlockSpec((1, 1), lambda t: (0, 0))],
        out_specs=pl.BlockSpec((B, 1), lambda t: (0, 0)),
        out_shape=jax.ShapeDtypeStruct((B, 1), f32),
        scratch_shapes=[pltpu.VMEM((B, B), f32), pltpu.VMEM((B, 1), f32)],
    )(s3a, s3b, s3c, s3d, degc, h2,
      W3.astype(f32).T, b3.astype(f32).reshape(1, 128), gidp,
      concat_o, Wr.astype(f32)[:, :128].T, Wr.astype(f32)[:, 128:].T,
      br.astype(f32).reshape(1, 1))
    return out


# speedup vs baseline: 11.3419x; 1.0002x over previous
import functools

import jax
import jax.numpy as jnp
from jax import lax
from jax.experimental import pallas as pl
from jax.experimental.pallas import tpu as pltpu
from jax.experimental.pallas import tpu_sc as plsc

N_NODES = 10000
N_EDGES = 320000
NPAD = 10240
ROWS_PER_TILE = NPAD // 16
EPAD = 327680
CHUNKS = EPAD // (16 * 128)
B = 128
T = 200
H = 64

_MESH = plsc.VectorSubcoreMesh(core_axis_name="c", subcore_axis_name="s")


def _zero_vec_ref(ref, n):
    def body(i, _):
        ref[pl.ds(i * 16, 16)] = jnp.zeros((16,), jnp.float32)
        return 0
    lax.fori_loop(0, n // 16, body, 0)


@functools.partial(
    pl.kernel,
    out_type=(jax.ShapeDtypeStruct((NPAD,), jnp.float32),
              jax.ShapeDtypeStruct((NPAD,), jnp.float32)),
    mesh=_MESH,
    compiler_params=pltpu.CompilerParams(use_tc_tiling_on_sc=False),
    scratch_types=[
        pltpu.VMEM((CHUNKS, 128), jnp.int32),
        pltpu.VMEM((CHUNKS, 128), jnp.int32),
        pltpu.VMEM((4, 128), jnp.float32),
        pltpu.VMEM((128,), jnp.float32),
        pltpu.VMEM((ROWS_PER_TILE,), jnp.float32),
        pltpu.VMEM_SHARED((NPAD,), jnp.float32),
        pltpu.VMEM_SHARED((NPAD,), jnp.float32),
        [pltpu.SemaphoreType.DMA] * 4,
        [pltpu.SemaphoreType.DMA] * 4,
    ],
)
def _deg_s1_kernel(srci, dsti, deg_o, s1_o,
                   srcv, dstv, valb, onesb, zb, deg_sh, s1_sh, gsem, ssem):
    c = lax.axis_index("c")

    @pl.when(c == 0)
    def _core0():
        s = lax.axis_index("s")
        _zero_vec_ref(zb, ROWS_PER_TILE)

        def ones16(i, _):
            onesb[pl.ds(i * 16, 16)] = jnp.ones((16,), jnp.float32)
            return 0
        lax.fori_loop(0, 8, ones16, 0)

        pltpu.sync_copy(zb, deg_sh.at[pl.ds(s * ROWS_PER_TILE, ROWS_PER_TILE)])
        pltpu.sync_copy(zb, s1_sh.at[pl.ds(s * ROWS_PER_TILE, ROWS_PER_TILE)])
        pltpu.sync_copy(srci.at[s], srcv)
        pltpu.sync_copy(dsti.at[s], dstv)
        plsc.subcore_barrier()

        def ph1(ck, _):
            pltpu.async_copy(onesb, deg_sh.at[dstv.at[ck]], ssem[0], add=True)
            return 0
        lax.fori_loop(0, CHUNKS, ph1, 0)

        def ph1w(ck, _):
            pltpu.make_async_copy(onesb, deg_sh.at[dstv.at[ck]],
                                  ssem[0]).wait()
            return 0
        lax.fori_loop(0, CHUNKS, ph1w, 0)
        plsc.subcore_barrier()

        sl = pl.ds(s * ROWS_PER_TILE, ROWS_PER_TILE)
        pltpu.sync_copy(deg_sh.at[sl], deg_o.at[sl])
        plsc.subcore_barrier()

        nb = 4
        rounds = CHUNKS // nb
        for b in range(nb):
            pltpu.async_copy(deg_o.at[srcv.at[b]], valb.at[b], gsem[b])

        def ph2(r, _):
            for b in range(nb):
                ck = r * nb + b
                pltpu.make_async_copy(
                    deg_o.at[srcv.at[ck]], valb.at[b], gsem[b]).wait()
                pltpu.async_copy(valb.at[b], s1_sh.at[dstv.at[ck]],
                                 ssem[b], add=True)

            @pl.when(r < rounds - 1)
            def _next():
                for b in range(nb):
                    ck = r * nb + b
                    pltpu.make_async_copy(
                        valb.at[b], s1_sh.at[dstv.at[ck]], ssem[b]).wait()
                    pltpu.async_copy(deg_o.at[srcv.at[ck + nb]],
                                     valb.at[b], gsem[b])
            return 0
        lax.fori_loop(0, rounds, ph2, 0)
        for b in range(nb):
            ck = CHUNKS - nb + b
            pltpu.make_async_copy(
                valb.at[b], s1_sh.at[dstv.at[ck]], ssem[b]).wait()
        plsc.subcore_barrier()
        pltpu.sync_copy(s1_sh.at[sl], s1_o.at[sl])


def _make_agg(dh):
    @functools.partial(
        pl.kernel,
        out_type=(jax.ShapeDtypeStruct((NPAD, dh), jnp.float32),
                  jax.ShapeDtypeStruct((NPAD, dh), jnp.float32)),
        mesh=_MESH,
        compiler_params=pltpu.CompilerParams(use_tc_tiling_on_sc=False),
        scratch_types=[
            pltpu.VMEM((CHUNKS, 128), jnp.int32),
            pltpu.VMEM((CHUNKS, 128), jnp.int32),
            pltpu.VMEM((4, 128, dh), jnp.float32),
            pltpu.VMEM_SHARED((NPAD, dh), jnp.float32),
            [pltpu.SemaphoreType.DMA] * 4,
            [pltpu.SemaphoreType.DMA] * 4,
        ],
    )
    def agg(ha, hb, srci, dsti, sa_o, sb_o, srcv, dstv, gbuf, s_sh,
            gsem, ssem):
        c = lax.axis_index("c")
        s = lax.axis_index("s")

        def zr(i, _):
            def zc(j, _):
                gbuf[0, i, pl.ds(j * 16, 16)] = jnp.zeros((16,), jnp.float32)
                return 0
            lax.fori_loop(0, dh // 16, zc, 0)
            return 0
        lax.fori_loop(0, 128, zr, 0)
        for r in range(ROWS_PER_TILE // 128):
            pltpu.sync_copy(gbuf.at[0],
                            s_sh.at[pl.ds(s * ROWS_PER_TILE + r * 128, 128)])
        pltpu.sync_copy(srci.at[s], srcv)
        pltpu.sync_copy(dsti.at[s], dstv)
        plsc.subcore_barrier()

        def run(h_hbm):
            nb = 4
            rounds = CHUNKS // nb
            for b in range(nb):
                pltpu.async_copy(h_hbm.at[srcv.at[b]], gbuf.at[b], gsem[b])

            def rnd(r, _):
                for b in range(nb):
                    ck = r * nb + b
                    pltpu.make_async_copy(
                        h_hbm.at[srcv.at[ck]], gbuf.at[b], gsem[b]).wait()
                    pltpu.async_copy(gbuf.at[b], s_sh.at[dstv.at[ck]],
                                     ssem[b], add=True)

                @pl.when(r < rounds - 1)
                def _next():
                    for b in range(nb):
                        ck = r * nb + b
                        pltpu.make_async_copy(
                            gbuf.at[b], s_sh.at[dstv.at[ck]], ssem[b]).wait()
                        pltpu.async_copy(h_hbm.at[srcv.at[ck + nb]],
                                         gbuf.at[b], gsem[b])
                return 0
            lax.fori_loop(0, rounds, rnd, 0)
            for b in range(nb):
                ck = CHUNKS - nb + b
                pltpu.make_async_copy(
                    gbuf.at[b], s_sh.at[dstv.at[ck]], ssem[b]).wait()

        @pl.when(c == 0)
        def _():
            run(ha)

        @pl.when(c == 1)
        def _():
            run(hb)

        plsc.subcore_barrier()
        sl = pl.ds(s * ROWS_PER_TILE, ROWS_PER_TILE)

        @pl.when(c == 0)
        def _():
            pltpu.sync_copy(s_sh.at[sl], sa_o.at[sl])

        @pl.when(c == 1)
        def _():
            pltpu.sync_copy(s_sh.at[sl], sb_o.at[sl])

    return agg


_agg64 = _make_agg(64)


def _layer1_body(deg_ref, s1_ref, w1_ref, b1_ref, out_ref):
    d = deg_ref[...]
    a = jnp.where(d > 0, s1_ref[...] / jnp.maximum(d, 1.0), d)
    out_ref[...] = jnp.maximum(a * w1_ref[...] + b1_ref[...], 0.0)


_RB = 1280
_NRB = NPAD // _RB


def _layer2_body(s2a_ref, s2b_ref, deg_ref, h1_ref, w2t_ref, b2_ref, out_ref):
    d = deg_ref[...]
    s2 = jnp.concatenate([s2a_ref[...], s2b_ref[...]], axis=1)
    a = jnp.where(d > 0, s2 / jnp.maximum(d, 1.0), h1_ref[...])
    out_ref[...] = jnp.maximum(
        jnp.dot(a, w2t_ref[...], preferred_element_type=jnp.float32,
                precision=lax.Precision.HIGHEST)
        + b2_ref[...], 0.0)


def _layer3_body(s3a_ref, s3b_ref, s3c_ref, s3d_ref, deg_ref, h2_ref,
                 w3t_ref, b3_ref, gid_ref,
                 co_ref, wra_ref, wrb_ref, br_ref, out_ref,
                 pool_ref, cnt_ref):
    t = pl.program_id(0)
    prec = lax.Precision.HIGHEST

    @pl.when(t == 0)
    def _init():
        pool_ref[...] = jnp.zeros((B, B), jnp.float32)
        cnt_ref[...] = jnp.zeros((B, 1), jnp.float32)

    d = deg_ref[...]
    s3 = jnp.concatenate([s3a_ref[...], s3b_ref[...], s3c_ref[...],
                          s3d_ref[...]], axis=1)
    a = jnp.where(d > 0, s3 / jnp.maximum(d, 1.0), h2_ref[...])
    h3 = jnp.maximum(
        jnp.dot(a, w3t_ref[...], preferred_element_type=jnp.float32,
                precision=prec)
        + b3_ref[...], 0.0)
    onehot = (gid_ref[...] == lax.broadcasted_iota(jnp.int32, (_RB, B), 1)
              ).astype(jnp.float32)
    dims = (((0,), (0,)), ((), ()))
    pool_ref[...] += lax.dot_general(onehot, h3, dims,
                                     preferred_element_type=jnp.float32,
                                     precision=prec)
    cnt_ref[...] += lax.dot_general(onehot, jnp.ones((_RB, 1), jnp.float32),
                                    dims, preferred_element_type=jnp.float32,
                                    precision=prec)

    @pl.when(t == _NRB - 1)
    def _fin():
        gmean = pool_ref[...] / jnp.maximum(cnt_ref[...], 1.0)
        out_ref[...] = (
            jnp.dot(co_ref[...], wra_ref[...],
                    preferred_element_type=jnp.float32, precision=prec)
            + jnp.dot(gmean, wrb_ref[...],
                      preferred_element_type=jnp.float32, precision=prec)
            + br_ref[...])


def _lstm_body(seq_ref, embp_ref, wft_ref, wbt_ref, bfull_ref,
               whh_ref, lo_ref, hi_ref, out_ref, tab_ref, h_ref, c_ref):
    t = pl.program_id(0)

    @pl.when(t == 0)
    def _init():
        tf = jnp.dot(embp_ref[...], wft_ref[...],
                     preferred_element_type=jnp.float32)
        tb = jnp.dot(embp_ref[...], wbt_ref[...],
                     preferred_element_type=jnp.float32)
        tab = jnp.concatenate([tf, tb], axis=0)
        th = tab.astype(jnp.bfloat16).astype(jnp.float32)
        tab_ref[...] = jnp.concatenate([th, tab - th], axis=0)
        h_ref[...] = jnp.zeros((2 * B, H), jnp.float32)
        c_ref[...] = jnp.zeros((2 * B, H), jnp.float32)

    isb = lax.broadcasted_iota(jnp.int32, (1, 2 * B), 1) >= B
    rowi = lax.broadcasted_iota(jnp.int32, (2 * B, 1), 0)
    mf = (rowi < B).astype(jnp.float32)
    dims = (((0,), (0,)), ((), ()))

    def step(tg, sv):
        tgt = sv + jnp.where(isb, 32, 0)
        ri = lax.broadcasted_iota(jnp.int32, (128, 2 * B), 0)
        oh_t = (jnp.logical_or(ri == tgt, ri == tgt + 64)
                ).astype(jnp.float32)
        h = h_ref[...]
        c = c_ref[...]
        hbig = jnp.concatenate([h * mf, h * (1.0 - mf)], axis=1)
        gates = (lax.dot_general(oh_t, tab_ref[...], dims,
                                 preferred_element_type=jnp.float32)
                 + jnp.dot(hbig, whh_ref[...],
                           preferred_element_type=jnp.float32)) + bfull_ref[...]
        i_g = gates[:, 0:H]
        f_g = gates[:, H:2 * H]
        g_g = gates[:, 2 * H:3 * H]
        o_g = gates[:, 3 * H:4 * H]
        c_new = jax.nn.sigmoid(f_g) * c + jax.nn.sigmoid(i_g) * jnp.tanh(g_g)
        h_new = jax.nn.sigmoid(o_g) * jnp.tanh(c_new)
        m = jnp.logical_and(tg >= lo_ref[...], tg < hi_ref[...])
        h_ref[...] = jnp.where(m, h_new, h)
        c_ref[...] = jnp.where(m, c_new, c)

    step(2 * t, seq_ref[0, 0:1, :])
    step(2 * t + 1, seq_ref[0, 1:2, :])

    @pl.when(t == T // 2 - 1)
    def _fin():
        out_ref[...] = h_ref[...]


def _run_lstm(seq2r, embp, wft, wbt, bfull, whh_cat, lo, hi):
    return pl.pallas_call(
        _lstm_body,
        grid=(T // 2,),
        in_specs=[
            pl.BlockSpec((1, 2, 2 * B), lambda t: (t, 0, 0)),
            pl.BlockSpec((32, 16), lambda t: (0, 0)),
            pl.BlockSpec((16, 4 * H), lambda t: (0, 0)),
            pl.BlockSpec((16, 4 * H), lambda t: (0, 0)),
            pl.BlockSpec((2 * B, 4 * H), lambda t: (0, 0)),
            pl.BlockSpec((2 * H, 4 * H), lambda t: (0, 0)),
            pl.BlockSpec((2 * B, 1), lambda t: (0, 0)),
            pl.BlockSpec((2 * B, 1), lambda t: (0, 0)),
        ],
        out_specs=pl.BlockSpec((2 * B, H), lambda t: (0, 0)),
        out_shape=jax.ShapeDtypeStruct((2 * B, H), jnp.float32),
        scratch_shapes=[
            pltpu.VMEM((128, 4 * H), jnp.float32),
            pltpu.VMEM((2 * B, H), jnp.float32),
            pltpu.VMEM((2 * B, H), jnp.float32),
        ],
    )(seq2r, embp, wft, wbt, bfull, whh_cat, lo, hi)


def kernel(seq, seq_len, edge_index, graph_ids, emb, Wih_f, Whh_f, b_f,
           Wih_b, Whh_b, b_b, W1, b1, W2, b2, W3, b3, Wr, br):
    f32 = jnp.float32

    npadd = EPAD - N_EDGES
    dummy = (N_NODES + (jnp.arange(npadd, dtype=jnp.int32) % (NPAD - N_NODES))
             ).astype(jnp.int32)
    src = jnp.concatenate([edge_index[0].astype(jnp.int32), dummy])
    dst = jnp.concatenate([edge_index[1].astype(jnp.int32), dummy])
    srci = src.reshape(16, CHUNKS, 128)
    dsti = dst.reshape(16, CHUNKS, 128)

    deg, s1 = _deg_s1_kernel(srci, dsti)
    degc = deg.reshape(NPAD, 1)

    h1 = pl.pallas_call(
        _layer1_body,
        out_shape=jax.ShapeDtypeStruct((NPAD, 128), f32),
    )(degc, s1.reshape(NPAD, 1), W1.astype(f32).reshape(1, 128),
      b1.astype(f32).reshape(1, 128))

    s2a, s2b = _agg64(h1[:, :64], h1[:, 64:], srci, dsti)

    rb = pl.BlockSpec((_RB, 64), lambda t: (t, 0))
    rb1 = pl.BlockSpec((_RB, 1), lambda t: (t, 0))
    rb128 = pl.BlockSpec((_RB, 128), lambda t: (t, 0))
    rb256 = pl.BlockSpec((_RB, 256), lambda t: (t, 0))
    h2 = pl.pallas_call(
        _layer2_body,
        grid=(_NRB,),
        in_specs=[rb, rb, rb1, rb128,
                  pl.BlockSpec((128, 256), lambda t: (0, 0)),
                  pl.BlockSpec((1, 256), lambda t: (0, 0))],
        out_specs=rb256,
        out_shape=jax.ShapeDtypeStruct((NPAD, 256), f32),
    )(s2a, s2b, degc, h1, W2.astype(f32).T, b2.astype(f32).reshape(1, 256))

    s3a, s3b = _agg64(h2[:, 0:64], h2[:, 64:128], srci, dsti)
    s3c, s3d = _agg64(h2[:, 128:192], h2[:, 192:256], srci, dsti)

    a = jnp.swapaxes(seq, 0, 1).astype(jnp.int32)
    seq2r = jnp.concatenate([a, a[::-1]], axis=1).reshape(T // 2, 2, 2 * B)
    embp = jnp.zeros((32, 16), f32).at[0:21, 0:10].set(emb.astype(f32))
    wft = jnp.zeros((16, 4 * H), f32).at[0:10].set(Wih_f.astype(f32).T)
    wbt = jnp.zeros((16, 4 * H), f32).at[0:10].set(Wih_b.astype(f32).T)
    sl = seq_len.astype(jnp.int32)
    lo = jnp.concatenate([jnp.zeros((B,), jnp.int32), T - sl]).reshape(2 * B, 1)
    hi = jnp.concatenate([sl, jnp.full((B,), T, jnp.int32)]).reshape(2 * B, 1)
    whh_cat = jnp.concatenate(
        [Whh_f.astype(f32).T, Whh_b.astype(f32).T], axis=0)
    bfull = jnp.concatenate(
        [jnp.broadcast_to(b_f.astype(f32), (B, 4 * H)),
         jnp.broadcast_to(b_b.astype(f32), (B, 4 * H))], axis=0)
    hout = _run_lstm(seq2r, embp, wft, wbt, bfull, whh_cat, lo, hi)
    concat_o = jnp.concatenate([hout[:B], hout[B:]], axis=1)

    gidp = jnp.concatenate(
        [graph_ids.astype(jnp.int32),
         jnp.full((NPAD - N_NODES,), 1 << 29, jnp.int32)]).reshape(NPAD, 1)
    out = pl.pallas_call(
        _layer3_body,
        grid=(_NRB,),
        in_specs=[rb, rb, rb, rb, rb1, rb256,
                  pl.BlockSpec((256, 128), lambda t: (0, 0)),
                  pl.BlockSpec((1, 128), lambda t: (0, 0)),
                  pl.BlockSpec((_RB, 1), lambda t: (t, 0)),
                  pl.BlockSpec((B, B), lambda t: (0, 0)),
                  pl.BlockSpec((B, 1), lambda t: (0, 0)),
                  pl.BlockSpec((B, 1), lambda t: (0, 0)),
                  pl.BlockSpec((1, 1), lambda t: (0, 0))],
        out_specs=pl.BlockSpec((B, 1), lambda t: (0, 0)),
        out_shape=jax.ShapeDtypeStruct((B, 1), f32),
        scratch_shapes=[pltpu.VMEM((B, B), f32), pltpu.VMEM((B, 1), f32)],
    )(s3a, s3b, s3c, s3d, degc, h2,
      W3.astype(f32).T, b3.astype(f32).reshape(1, 128), gidp,
      concat_o, Wr.astype(f32)[:, :128].T, Wr.astype(f32)[:, 128:].T,
      br.astype(f32).reshape(1, 1))
    return out


# speedup vs baseline: 11.4970x; 1.0137x over previous
import functools

import jax
import jax.numpy as jnp
from jax import lax
from jax.experimental import pallas as pl
from jax.experimental.pallas import tpu as pltpu
from jax.experimental.pallas import tpu_sc as plsc

N_NODES = 10000
N_EDGES = 320000
NPAD = 10240
ROWS_PER_TILE = NPAD // 16
EPAD = 327680
CHUNKS = EPAD // (16 * 128)
B = 128
T = 200
H = 64

_MESH = plsc.VectorSubcoreMesh(core_axis_name="c", subcore_axis_name="s")


def _zero_vec_ref(ref, n):
    def body(i, _):
        ref[pl.ds(i * 16, 16)] = jnp.zeros((16,), jnp.float32)
        return 0
    lax.fori_loop(0, n // 16, body, 0)


@functools.partial(
    pl.kernel,
    out_type=(jax.ShapeDtypeStruct((NPAD,), jnp.float32),
              jax.ShapeDtypeStruct((NPAD,), jnp.float32)),
    mesh=_MESH,
    compiler_params=pltpu.CompilerParams(use_tc_tiling_on_sc=False),
    scratch_types=[
        pltpu.VMEM((CHUNKS, 128), jnp.int32),
        pltpu.VMEM((CHUNKS, 128), jnp.int32),
        pltpu.VMEM((4, 128), jnp.float32),
        pltpu.VMEM((128,), jnp.float32),
        pltpu.VMEM((ROWS_PER_TILE,), jnp.float32),
        pltpu.VMEM_SHARED((NPAD,), jnp.float32),
        pltpu.VMEM_SHARED((NPAD,), jnp.float32),
        [pltpu.SemaphoreType.DMA] * 4,
        [pltpu.SemaphoreType.DMA] * 4,
    ],
)
def _deg_s1_kernel(srci, dsti, deg_o, s1_o,
                   srcv, dstv, valb, onesb, zb, deg_sh, s1_sh, gsem, ssem):
    c = lax.axis_index("c")

    @pl.when(c == 0)
    def _core0():
        s = lax.axis_index("s")
        _zero_vec_ref(zb, ROWS_PER_TILE)

        def ones16(i, _):
            onesb[pl.ds(i * 16, 16)] = jnp.ones((16,), jnp.float32)
            return 0
        lax.fori_loop(0, 8, ones16, 0)

        pltpu.sync_copy(zb, deg_sh.at[pl.ds(s * ROWS_PER_TILE, ROWS_PER_TILE)])
        pltpu.sync_copy(zb, s1_sh.at[pl.ds(s * ROWS_PER_TILE, ROWS_PER_TILE)])
        pltpu.sync_copy(srci.at[s], srcv)
        pltpu.sync_copy(dsti.at[s], dstv)
        plsc.subcore_barrier()

        def ph1(ck, _):
            pltpu.async_copy(onesb, deg_sh.at[dstv.at[ck]], ssem[0], add=True)
            return 0
        lax.fori_loop(0, CHUNKS, ph1, 0)

        def ph1w(ck, _):
            pltpu.make_async_copy(onesb, deg_sh.at[dstv.at[ck]],
                                  ssem[0]).wait()
            return 0
        lax.fori_loop(0, CHUNKS, ph1w, 0)
        plsc.subcore_barrier()

        sl = pl.ds(s * ROWS_PER_TILE, ROWS_PER_TILE)
        pltpu.sync_copy(deg_sh.at[sl], deg_o.at[sl])
        plsc.subcore_barrier()

        nb = 4
        rounds = CHUNKS // nb
        for b in range(nb):
            pltpu.async_copy(deg_o.at[srcv.at[b]], valb.at[b], gsem[b])

        def ph2(r, _):
            for b in range(nb):
                ck = r * nb + b
                pltpu.make_async_copy(
                    deg_o.at[srcv.at[ck]], valb.at[b], gsem[b]).wait()
                pltpu.async_copy(valb.at[b], s1_sh.at[dstv.at[ck]],
                                 ssem[b], add=True)

            @pl.when(r < rounds - 1)
            def _next():
                for b in range(nb):
                    ck = r * nb + b
                    pltpu.make_async_copy(
                        valb.at[b], s1_sh.at[dstv.at[ck]], ssem[b]).wait()
                    pltpu.async_copy(deg_o.at[srcv.at[ck + nb]],
                                     valb.at[b], gsem[b])
            return 0
        lax.fori_loop(0, rounds, ph2, 0)
        for b in range(nb):
            ck = CHUNKS - nb + b
            pltpu.make_async_copy(
                valb.at[b], s1_sh.at[dstv.at[ck]], ssem[b]).wait()
        plsc.subcore_barrier()
        pltpu.sync_copy(s1_sh.at[sl], s1_o.at[sl])


def _make_agg(dh):
    @functools.partial(
        pl.kernel,
        out_type=(jax.ShapeDtypeStruct((NPAD, dh), jnp.float32),
                  jax.ShapeDtypeStruct((NPAD, dh), jnp.float32)),
        mesh=_MESH,
        compiler_params=pltpu.CompilerParams(use_tc_tiling_on_sc=False),
        scratch_types=[
            pltpu.VMEM((CHUNKS, 128), jnp.int32),
            pltpu.VMEM((CHUNKS, 128), jnp.int32),
            pltpu.VMEM((5, 128, dh), jnp.float32),
            pltpu.VMEM_SHARED((NPAD, dh), jnp.float32),
            [pltpu.SemaphoreType.DMA] * 5,
            [pltpu.SemaphoreType.DMA] * 5,
        ],
    )
    def agg(ha, hb, srci, dsti, sa_o, sb_o, srcv, dstv, gbuf, s_sh,
            gsem, ssem):
        c = lax.axis_index("c")
        s = lax.axis_index("s")

        def zr(i, _):
            def zc(j, _):
                gbuf[0, i, pl.ds(j * 16, 16)] = jnp.zeros((16,), jnp.float32)
                return 0
            lax.fori_loop(0, dh // 16, zc, 0)
            return 0
        lax.fori_loop(0, 128, zr, 0)
        for r in range(ROWS_PER_TILE // 128):
            pltpu.sync_copy(gbuf.at[0],
                            s_sh.at[pl.ds(s * ROWS_PER_TILE + r * 128, 128)])
        pltpu.sync_copy(srci.at[s], srcv)
        pltpu.sync_copy(dsti.at[s], dstv)
        plsc.subcore_barrier()

        def run(h_hbm):
            nb = 5
            rounds = CHUNKS // nb
            for b in range(nb):
                pltpu.async_copy(h_hbm.at[srcv.at[b]], gbuf.at[b], gsem[b])

            def rnd(r, _):
                for b in range(nb):
                    ck = r * nb + b
                    pltpu.make_async_copy(
                        h_hbm.at[srcv.at[ck]], gbuf.at[b], gsem[b]).wait()
                    pltpu.async_copy(gbuf.at[b], s_sh.at[dstv.at[ck]],
                                     ssem[b], add=True)

                @pl.when(r < rounds - 1)
                def _next():
                    for b in range(nb):
                        ck = r * nb + b
                        pltpu.make_async_copy(
                            gbuf.at[b], s_sh.at[dstv.at[ck]], ssem[b]).wait()
                        pltpu.async_copy(h_hbm.at[srcv.at[ck + nb]],
                                         gbuf.at[b], gsem[b])
                return 0
            lax.fori_loop(0, rounds, rnd, 0)
            for b in range(nb):
                ck = CHUNKS - nb + b
                pltpu.make_async_copy(
                    gbuf.at[b], s_sh.at[dstv.at[ck]], ssem[b]).wait()

        @pl.when(c == 0)
        def _():
            run(ha)

        @pl.when(c == 1)
        def _():
            run(hb)

        plsc.subcore_barrier()
        sl = pl.ds(s * ROWS_PER_TILE, ROWS_PER_TILE)

        @pl.when(c == 0)
        def _():
            pltpu.sync_copy(s_sh.at[sl], sa_o.at[sl])

        @pl.when(c == 1)
        def _():
            pltpu.sync_copy(s_sh.at[sl], sb_o.at[sl])

    return agg


_agg64 = _make_agg(64)


def _layer1_body(deg_ref, s1_ref, w1_ref, b1_ref, out_ref):
    d = deg_ref[...]
    a = jnp.where(d > 0, s1_ref[...] / jnp.maximum(d, 1.0), d)
    out_ref[...] = jnp.maximum(a * w1_ref[...] + b1_ref[...], 0.0)


_RB = 1280
_NRB = NPAD // _RB


def _layer2_body(s2a_ref, s2b_ref, deg_ref, h1_ref, w2t_ref, b2_ref, out_ref):
    d = deg_ref[...]
    s2 = jnp.concatenate([s2a_ref[...], s2b_ref[...]], axis=1)
    a = jnp.where(d > 0, s2 / jnp.maximum(d, 1.0), h1_ref[...])
    out_ref[...] = jnp.maximum(
        jnp.dot(a, w2t_ref[...], preferred_element_type=jnp.float32,
                precision=lax.Precision.HIGHEST)
        + b2_ref[...], 0.0)


def _layer3_body(s3a_ref, s3b_ref, s3c_ref, s3d_ref, deg_ref, h2_ref,
                 w3t_ref, b3_ref, gid_ref,
                 co_ref, wra_ref, wrb_ref, br_ref, out_ref,
                 pool_ref, cnt_ref):
    t = pl.program_id(0)
    prec = lax.Precision.HIGHEST

    @pl.when(t == 0)
    def _init():
        pool_ref[...] = jnp.zeros((B, B), jnp.float32)
        cnt_ref[...] = jnp.zeros((B, 1), jnp.float32)

    d = deg_ref[...]
    s3 = jnp.concatenate([s3a_ref[...], s3b_ref[...], s3c_ref[...],
                          s3d_ref[...]], axis=1)
    a = jnp.where(d > 0, s3 / jnp.maximum(d, 1.0), h2_ref[...])
    h3 = jnp.maximum(
        jnp.dot(a, w3t_ref[...], preferred_element_type=jnp.float32,
                precision=prec)
        + b3_ref[...], 0.0)
    onehot = (gid_ref[...] == lax.broadcasted_iota(jnp.int32, (_RB, B), 1)
              ).astype(jnp.float32)
    dims = (((0,), (0,)), ((), ()))
    pool_ref[...] += lax.dot_general(onehot, h3, dims,
                                     preferred_element_type=jnp.float32,
                                     precision=prec)
    cnt_ref[...] += lax.dot_general(onehot, jnp.ones((_RB, 1), jnp.float32),
                                    dims, preferred_element_type=jnp.float32,
                                    precision=prec)

    @pl.when(t == _NRB - 1)
    def _fin():
        gmean = pool_ref[...] / jnp.maximum(cnt_ref[...], 1.0)
        out_ref[...] = (
            jnp.dot(co_ref[...], wra_ref[...],
                    preferred_element_type=jnp.float32, precision=prec)
            + jnp.dot(gmean, wrb_ref[...],
                      preferred_element_type=jnp.float32, precision=prec)
            + br_ref[...])


def _lstm_body(seq_ref, embp_ref, wft_ref, wbt_ref, bfull_ref,
               whh_ref, lo_ref, hi_ref, out_ref, tab_ref, h_ref, c_ref):
    t = pl.program_id(0)

    @pl.when(t == 0)
    def _init():
        tf = jnp.dot(embp_ref[...], wft_ref[...],
                     preferred_element_type=jnp.float32)
        tb = jnp.dot(embp_ref[...], wbt_ref[...],
                     preferred_element_type=jnp.float32)
        tab = jnp.concatenate([tf, tb], axis=0)
        th = tab.astype(jnp.bfloat16).astype(jnp.float32)
        tab_ref[...] = jnp.concatenate([th, tab - th], axis=0)
        h_ref[...] = jnp.zeros((2 * B, H), jnp.float32)
        c_ref[...] = jnp.zeros((2 * B, H), jnp.float32)

    isb = lax.broadcasted_iota(jnp.int32, (1, 2 * B), 1) >= B
    rowi = lax.broadcasted_iota(jnp.int32, (2 * B, 1), 0)
    mf = (rowi < B).astype(jnp.float32)
    dims = (((0,), (0,)), ((), ()))

    def step(tg, sv):
        tgt = sv + jnp.where(isb, 32, 0)
        ri = lax.broadcasted_iota(jnp.int32, (128, 2 * B), 0)
        oh_t = (jnp.logical_or(ri == tgt, ri == tgt + 64)
                ).astype(jnp.float32)
        h = h_ref[...]
        c = c_ref[...]
        hbig = jnp.concatenate([h * mf, h * (1.0 - mf)], axis=1)
        gates = (lax.dot_general(oh_t, tab_ref[...], dims,
                                 preferred_element_type=jnp.float32)
                 + jnp.dot(hbig, whh_ref[...],
                           preferred_element_type=jnp.float32)) + bfull_ref[...]
        i_g = gates[:, 0:H]
        f_g = gates[:, H:2 * H]
        g_g = gates[:, 2 * H:3 * H]
        o_g = gates[:, 3 * H:4 * H]
        c_new = jax.nn.sigmoid(f_g) * c + jax.nn.sigmoid(i_g) * jnp.tanh(g_g)
        h_new = jax.nn.sigmoid(o_g) * jnp.tanh(c_new)
        m = jnp.logical_and(tg >= lo_ref[...], tg < hi_ref[...])
        h_ref[...] = jnp.where(m, h_new, h)
        c_ref[...] = jnp.where(m, c_new, c)

    step(2 * t, seq_ref[0, 0:1, :])
    step(2 * t + 1, seq_ref[0, 1:2, :])

    @pl.when(t == T // 2 - 1)
    def _fin():
        out_ref[...] = h_ref[...]


def _run_lstm(seq2r, embp, wft, wbt, bfull, whh_cat, lo, hi):
    return pl.pallas_call(
        _lstm_body,
        grid=(T // 2,),
        in_specs=[
            pl.BlockSpec((1, 2, 2 * B), lambda t: (t, 0, 0)),
            pl.BlockSpec((32, 16), lambda t: (0, 0)),
            pl.BlockSpec((16, 4 * H), lambda t: (0, 0)),
            pl.BlockSpec((16, 4 * H), lambda t: (0, 0)),
            pl.BlockSpec((2 * B, 4 * H), lambda t: (0, 0)),
            pl.BlockSpec((2 * H, 4 * H), lambda t: (0, 0)),
            pl.BlockSpec((2 * B, 1), lambda t: (0, 0)),
            pl.BlockSpec((2 * B, 1), lambda t: (0, 0)),
        ],
        out_specs=pl.BlockSpec((2 * B, H), lambda t: (0, 0)),
        out_shape=jax.ShapeDtypeStruct((2 * B, H), jnp.float32),
        scratch_shapes=[
            pltpu.VMEM((128, 4 * H), jnp.float32),
            pltpu.VMEM((2 * B, H), jnp.float32),
            pltpu.VMEM((2 * B, H), jnp.float32),
        ],
    )(seq2r, embp, wft, wbt, bfull, whh_cat, lo, hi)


def kernel(seq, seq_len, edge_index, graph_ids, emb, Wih_f, Whh_f, b_f,
           Wih_b, Whh_b, b_b, W1, b1, W2, b2, W3, b3, Wr, br):
    f32 = jnp.float32

    npadd = EPAD - N_EDGES
    dummy = (N_NODES + (jnp.arange(npadd, dtype=jnp.int32) % (NPAD - N_NODES))
             ).astype(jnp.int32)
    src = jnp.concatenate([edge_index[0].astype(jnp.int32), dummy])
    dst = jnp.concatenate([edge_index[1].astype(jnp.int32), dummy])
    srci = src.reshape(16, CHUNKS, 128)
    dsti = dst.reshape(16, CHUNKS, 128)

    deg, s1 = _deg_s1_kernel(srci, dsti)
    degc = deg.reshape(NPAD, 1)

    h1 = pl.pallas_call(
        _layer1_body,
        out_shape=jax.ShapeDtypeStruct((NPAD, 128), f32),
    )(degc, s1.reshape(NPAD, 1), W1.astype(f32).reshape(1, 128),
      b1.astype(f32).reshape(1, 128))

    s2a, s2b = _agg64(h1[:, :64], h1[:, 64:], srci, dsti)

    rb = pl.BlockSpec((_RB, 64), lambda t: (t, 0))
    rb1 = pl.BlockSpec((_RB, 1), lambda t: (t, 0))
    rb128 = pl.BlockSpec((_RB, 128), lambda t: (t, 0))
    rb256 = pl.BlockSpec((_RB, 256), lambda t: (t, 0))
    h2 = pl.pallas_call(
        _layer2_body,
        grid=(_NRB,),
        in_specs=[rb, rb, rb1, rb128,
                  pl.BlockSpec((128, 256), lambda t: (0, 0)),
                  pl.BlockSpec((1, 256), lambda t: (0, 0))],
        out_specs=rb256,
        out_shape=jax.ShapeDtypeStruct((NPAD, 256), f32),
    )(s2a, s2b, degc, h1, W2.astype(f32).T, b2.astype(f32).reshape(1, 256))

    s3a, s3b = _agg64(h2[:, 0:64], h2[:, 64:128], srci, dsti)
    s3c, s3d = _agg64(h2[:, 128:192], h2[:, 192:256], srci, dsti)

    a = jnp.swapaxes(seq, 0, 1).astype(jnp.int32)
    seq2r = jnp.concatenate([a, a[::-1]], axis=1).reshape(T // 2, 2, 2 * B)
    embp = jnp.zeros((32, 16), f32).at[0:21, 0:10].set(emb.astype(f32))
    wft = jnp.zeros((16, 4 * H), f32).at[0:10].set(Wih_f.astype(f32).T)
    wbt = jnp.zeros((16, 4 * H), f32).at[0:10].set(Wih_b.astype(f32).T)
    sl = seq_len.astype(jnp.int32)
    lo = jnp.concatenate([jnp.zeros((B,), jnp.int32), T - sl]).reshape(2 * B, 1)
    hi = jnp.concatenate([sl, jnp.full((B,), T, jnp.int32)]).reshape(2 * B, 1)
    whh_cat = jnp.concatenate(
        [Whh_f.astype(f32).T, Whh_b.astype(f32).T], axis=0)
    bfull = jnp.concatenate(
        [jnp.broadcast_to(b_f.astype(f32), (B, 4 * H)),
         jnp.broadcast_to(b_b.astype(f32), (B, 4 * H))], axis=0)
    hout = _run_lstm(seq2r, embp, wft, wbt, bfull, whh_cat, lo, hi)
    concat_o = jnp.concatenate([hout[:B], hout[B:]], axis=1)

    gidp = jnp.concatenate(
        [graph_ids.astype(jnp.int32),
         jnp.full((NPAD - N_NODES,), 1 << 29, jnp.int32)]).reshape(NPAD, 1)
    out = pl.pallas_call(
        _layer3_body,
        grid=(_NRB,),
        in_specs=[rb, rb, rb, rb, rb1, rb256,
                  pl.BlockSpec((256, 128), lambda t: (0, 0)),
                  pl.BlockSpec((1, 128), lambda t: (0, 0)),
                  pl.BlockSpec((_RB, 1), lambda t: (t, 0)),
                  pl.BlockSpec((B, B), lambda t: (0, 0)),
                  pl.BlockSpec((B, 1), lambda t: (0, 0)),
                  pl.BlockSpec((B, 1), lambda t: (0, 0)),
                  pl.BlockSpec((1, 1), lambda t: (0, 0))],
        out_specs=pl.BlockSpec((B, 1), lambda t: (0, 0)),
        out_shape=jax.ShapeDtypeStruct((B, 1), f32),
        scratch_shapes=[pltpu.VMEM((B, B), f32), pltpu.VMEM((B, 1), f32)],
    )(s3a, s3b, s3c, s3d, degc, h2,
      W3.astype(f32).T, b3.astype(f32).reshape(1, 128), gidp,
      concat_o, Wr.astype(f32)[:, :128].T, Wr.astype(f32)[:, 128:].T,
      br.astype(f32).reshape(1, 1))
    return out


# speedup vs baseline: 13.2135x; 1.1493x over previous
import functools

import jax
import jax.numpy as jnp
from jax import lax
from jax.experimental import pallas as pl
from jax.experimental.pallas import tpu as pltpu
from jax.experimental.pallas import tpu_sc as plsc

N_NODES = 10000
N_EDGES = 320000
NPAD = 10240
ROWS_PER_TILE = NPAD // 16
EPAD = 327680
CHUNKS = EPAD // (16 * 128)
B = 128
T = 200
H = 64

_MESH = plsc.VectorSubcoreMesh(core_axis_name="c", subcore_axis_name="s")


def _zero_vec_ref(ref, n):
    def body(i, _):
        ref[pl.ds(i * 16, 16)] = jnp.zeros((16,), jnp.float32)
        return 0
    lax.fori_loop(0, n // 16, body, 0)


@functools.partial(
    pl.kernel,
    out_type=(jax.ShapeDtypeStruct((NPAD,), jnp.float32),
              jax.ShapeDtypeStruct((NPAD,), jnp.float32)),
    mesh=_MESH,
    compiler_params=pltpu.CompilerParams(use_tc_tiling_on_sc=False),
    scratch_types=[
        pltpu.VMEM((CHUNKS, 128), jnp.int32),
        pltpu.VMEM((CHUNKS, 128), jnp.int32),
        pltpu.VMEM((4, 128), jnp.float32),
        pltpu.VMEM((128,), jnp.float32),
        pltpu.VMEM((ROWS_PER_TILE,), jnp.float32),
        pltpu.VMEM_SHARED((NPAD,), jnp.float32),
        pltpu.VMEM_SHARED((NPAD,), jnp.float32),
        [pltpu.SemaphoreType.DMA] * 4,
        [pltpu.SemaphoreType.DMA] * 4,
    ],
)
def _deg_s1_kernel(srci, dsti, deg_o, s1_o,
                   srcv, dstv, valb, onesb, zb, deg_sh, s1_sh, gsem, ssem):
    c = lax.axis_index("c")

    @pl.when(c == 0)
    def _core0():
        s = lax.axis_index("s")
        _zero_vec_ref(zb, ROWS_PER_TILE)

        def ones16(i, _):
            onesb[pl.ds(i * 16, 16)] = jnp.ones((16,), jnp.float32)
            return 0
        lax.fori_loop(0, 8, ones16, 0)

        pltpu.sync_copy(zb, deg_sh.at[pl.ds(s * ROWS_PER_TILE, ROWS_PER_TILE)])
        pltpu.sync_copy(zb, s1_sh.at[pl.ds(s * ROWS_PER_TILE, ROWS_PER_TILE)])
        pltpu.sync_copy(srci.at[s], srcv)
        pltpu.sync_copy(dsti.at[s], dstv)
        plsc.subcore_barrier()

        def ph1(ck, _):
            pltpu.async_copy(onesb, deg_sh.at[dstv.at[ck]], ssem[0], add=True)
            return 0
        lax.fori_loop(0, CHUNKS, ph1, 0)

        def ph1w(ck, _):
            pltpu.make_async_copy(onesb, deg_sh.at[dstv.at[ck]],
                                  ssem[0]).wait()
            return 0
        lax.fori_loop(0, CHUNKS, ph1w, 0)
        plsc.subcore_barrier()

        sl = pl.ds(s * ROWS_PER_TILE, ROWS_PER_TILE)
        pltpu.sync_copy(deg_sh.at[sl], deg_o.at[sl])
        plsc.subcore_barrier()

        nb = 4
        rounds = CHUNKS // nb
        for b in range(nb):
            pltpu.async_copy(deg_sh.at[srcv.at[b]], valb.at[b], gsem[b])

        def ph2(r, _):
            for b in range(nb):
                ck = r * nb + b
                pltpu.make_async_copy(
                    deg_sh.at[srcv.at[ck]], valb.at[b], gsem[b]).wait()
                pltpu.async_copy(valb.at[b], s1_sh.at[dstv.at[ck]],
                                 ssem[b], add=True)

            @pl.when(r < rounds - 1)
            def _next():
                for b in range(nb):
                    ck = r * nb + b
                    pltpu.make_async_copy(
                        valb.at[b], s1_sh.at[dstv.at[ck]], ssem[b]).wait()
                    pltpu.async_copy(deg_sh.at[srcv.at[ck + nb]],
                                     valb.at[b], gsem[b])
            return 0
        lax.fori_loop(0, rounds, ph2, 0)
        for b in range(nb):
            ck = CHUNKS - nb + b
            pltpu.make_async_copy(
                valb.at[b], s1_sh.at[dstv.at[ck]], ssem[b]).wait()
        plsc.subcore_barrier()
        pltpu.sync_copy(s1_sh.at[sl], s1_o.at[sl])


def _make_agg(dh):
    @functools.partial(
        pl.kernel,
        out_type=(jax.ShapeDtypeStruct((NPAD, dh), jnp.float32),
                  jax.ShapeDtypeStruct((NPAD, dh), jnp.float32)),
        mesh=_MESH,
        compiler_params=pltpu.CompilerParams(use_tc_tiling_on_sc=False),
        scratch_types=[
            pltpu.VMEM((CHUNKS, 128), jnp.int32),
            pltpu.VMEM((CHUNKS, 128), jnp.int32),
            pltpu.VMEM((5, 128, dh), jnp.float32),
            pltpu.VMEM_SHARED((NPAD, dh), jnp.float32),
            [pltpu.SemaphoreType.DMA] * 5,
            [pltpu.SemaphoreType.DMA] * 5,
        ],
    )
    def agg(ha, hb, srci, dsti, sa_o, sb_o, srcv, dstv, gbuf, s_sh,
            gsem, ssem):
        c = lax.axis_index("c")
        s = lax.axis_index("s")

        def zr(i, _):
            def zc(j, _):
                gbuf[0, i, pl.ds(j * 16, 16)] = jnp.zeros((16,), jnp.float32)
                return 0
            lax.fori_loop(0, dh // 16, zc, 0)
            return 0
        lax.fori_loop(0, 128, zr, 0)
        for r in range(ROWS_PER_TILE // 128):
            pltpu.sync_copy(gbuf.at[0],
                            s_sh.at[pl.ds(s * ROWS_PER_TILE + r * 128, 128)])
        pltpu.sync_copy(srci.at[s], srcv)
        pltpu.sync_copy(dsti.at[s], dstv)
        plsc.subcore_barrier()

        def run(h_hbm):
            nb = 5
            rounds = CHUNKS // nb
            for b in range(nb):
                pltpu.async_copy(h_hbm.at[srcv.at[b]], gbuf.at[b], gsem[b])

            def rnd(r, _):
                for b in range(nb):
                    ck = r * nb + b
                    pltpu.make_async_copy(
                        h_hbm.at[srcv.at[ck]], gbuf.at[b], gsem[b]).wait()
                    pltpu.async_copy(gbuf.at[b], s_sh.at[dstv.at[ck]],
                                     ssem[b], add=True)

                @pl.when(r < rounds - 1)
                def _next():
                    for b in range(nb):
                        ck = r * nb + b
                        pltpu.make_async_copy(
                            gbuf.at[b], s_sh.at[dstv.at[ck]], ssem[b]).wait()
                        pltpu.async_copy(h_hbm.at[srcv.at[ck + nb]],
                                         gbuf.at[b], gsem[b])
                return 0
            lax.fori_loop(0, rounds, rnd, 0)
            for b in range(nb):
                ck = CHUNKS - nb + b
                pltpu.make_async_copy(
                    gbuf.at[b], s_sh.at[dstv.at[ck]], ssem[b]).wait()

        @pl.when(c == 0)
        def _():
            run(ha)

        @pl.when(c == 1)
        def _():
            run(hb)

        plsc.subcore_barrier()
        sl = pl.ds(s * ROWS_PER_TILE, ROWS_PER_TILE)

        @pl.when(c == 0)
        def _():
            pltpu.sync_copy(s_sh.at[sl], sa_o.at[sl])

        @pl.when(c == 1)
        def _():
            pltpu.sync_copy(s_sh.at[sl], sb_o.at[sl])

    return agg


_agg64 = _make_agg(64)


def _layer1_body(deg_ref, s1_ref, w1_ref, b1_ref, out_ref, oa_ref, ob_ref):
    d = deg_ref[...]
    a = jnp.where(d > 0, s1_ref[...] / jnp.maximum(d, 1.0), d)
    h1 = jnp.maximum(a * w1_ref[...] + b1_ref[...], 0.0)
    out_ref[...] = h1
    oa_ref[...] = h1[:, 0:64]
    ob_ref[...] = h1[:, 64:128]


_RB = 1280
_NRB = NPAD // _RB


def _layer2_body(s2a_ref, s2b_ref, deg_ref, h1_ref, w2t_ref, b2_ref, out_ref,
                 o0_ref, o1_ref, o2_ref, o3_ref):
    d = deg_ref[...]
    s2 = jnp.concatenate([s2a_ref[...], s2b_ref[...]], axis=1)
    a = jnp.where(d > 0, s2 / jnp.maximum(d, 1.0), h1_ref[...])
    h2 = jnp.maximum(
        jnp.dot(a, w2t_ref[...], preferred_element_type=jnp.float32)
        + b2_ref[...], 0.0)
    out_ref[...] = h2
    o0_ref[...] = h2[:, 0:64]
    o1_ref[...] = h2[:, 64:128]
    o2_ref[...] = h2[:, 128:192]
    o3_ref[...] = h2[:, 192:256]


def _layer3_body(s3a_ref, s3b_ref, s3c_ref, s3d_ref, deg_ref, h2_ref,
                 w3t_ref, b3_ref, gid_ref,
                 co_ref, wra_ref, wrb_ref, br_ref, out_ref,
                 pool_ref, cnt_ref):
    t = pl.program_id(0)

    @pl.when(t == 0)
    def _init():
        pool_ref[...] = jnp.zeros((B, B), jnp.float32)
        cnt_ref[...] = jnp.zeros((B, 1), jnp.float32)

    d = deg_ref[...]
    s3 = jnp.concatenate([s3a_ref[...], s3b_ref[...], s3c_ref[...],
                          s3d_ref[...]], axis=1)
    a = jnp.where(d > 0, s3 / jnp.maximum(d, 1.0), h2_ref[...])
    h3 = jnp.maximum(
        jnp.dot(a, w3t_ref[...], preferred_element_type=jnp.float32)
        + b3_ref[...], 0.0)
    onehot = (gid_ref[...] == lax.broadcasted_iota(jnp.int32, (_RB, B), 1)
              ).astype(jnp.float32)
    dims = (((0,), (0,)), ((), ()))
    h3h = h3.astype(jnp.bfloat16).astype(jnp.float32)
    pool_ref[...] += (
        lax.dot_general(onehot, h3h, dims, preferred_element_type=jnp.float32)
        + lax.dot_general(onehot, h3 - h3h, dims,
                          preferred_element_type=jnp.float32))
    cnt_ref[...] += lax.dot_general(onehot, jnp.ones((_RB, 1), jnp.float32),
                                    dims, preferred_element_type=jnp.float32)

    @pl.when(t == _NRB - 1)
    def _fin():
        gmean = pool_ref[...] / jnp.maximum(cnt_ref[...], 1.0)
        out_ref[...] = (
            jnp.dot(co_ref[...], wra_ref[...],
                    preferred_element_type=jnp.float32)
            + jnp.dot(gmean, wrb_ref[...],
                      preferred_element_type=jnp.float32)
            + br_ref[...])


def _lstm_body(seq_ref, embp_ref, wft_ref, wbt_ref, bfull_ref,
               whh_ref, lo_ref, hi_ref, out_ref, tab_ref, h_ref, c_ref):
    t = pl.program_id(0)

    @pl.when(t == 0)
    def _init():
        tf = jnp.dot(embp_ref[...], wft_ref[...],
                     preferred_element_type=jnp.float32)
        tb = jnp.dot(embp_ref[...], wbt_ref[...],
                     preferred_element_type=jnp.float32)
        tab = jnp.concatenate([tf, tb], axis=0)
        th = tab.astype(jnp.bfloat16).astype(jnp.float32)
        tab_ref[...] = jnp.concatenate([th, tab - th], axis=0)
        h_ref[...] = jnp.zeros((2 * B, H), jnp.float32)
        c_ref[...] = jnp.zeros((2 * B, H), jnp.float32)

    isb = lax.broadcasted_iota(jnp.int32, (1, 2 * B), 1) >= B
    rowi = lax.broadcasted_iota(jnp.int32, (2 * B, 1), 0)
    mf = (rowi < B).astype(jnp.float32)
    dims = (((0,), (0,)), ((), ()))

    def step(tg, sv):
        tgt = sv + jnp.where(isb, 32, 0)
        ri = lax.broadcasted_iota(jnp.int32, (128, 2 * B), 0)
        oh_t = (jnp.logical_or(ri == tgt, ri == tgt + 64)
                ).astype(jnp.float32)
        h = h_ref[...]
        c = c_ref[...]
        hbig = jnp.concatenate([h * mf, h * (1.0 - mf)], axis=1)
        gates = (lax.dot_general(oh_t, tab_ref[...], dims,
                                 preferred_element_type=jnp.float32)
                 + jnp.dot(hbig, whh_ref[...],
                           preferred_element_type=jnp.float32)) + bfull_ref[...]
        i_g = gates[:, 0:H]
        f_g = gates[:, H:2 * H]
        g_g = gates[:, 2 * H:3 * H]
        o_g = gates[:, 3 * H:4 * H]
        c_new = jax.nn.sigmoid(f_g) * c + jax.nn.sigmoid(i_g) * jnp.tanh(g_g)
        h_new = jax.nn.sigmoid(o_g) * jnp.tanh(c_new)
        m = jnp.logical_and(tg >= lo_ref[...], tg < hi_ref[...])
        h_ref[...] = jnp.where(m, h_new, h)
        c_ref[...] = jnp.where(m, c_new, c)

    step(2 * t, seq_ref[0, 0:1, :])
    step(2 * t + 1, seq_ref[0, 1:2, :])

    @pl.when(t == T // 2 - 1)
    def _fin():
        out_ref[...] = h_ref[...]


def _run_lstm(seq2r, embp, wft, wbt, bfull, whh_cat, lo, hi):
    return pl.pallas_call(
        _lstm_body,
        grid=(T // 2,),
        in_specs=[
            pl.BlockSpec((1, 2, 2 * B), lambda t: (t, 0, 0)),
            pl.BlockSpec((32, 16), lambda t: (0, 0)),
            pl.BlockSpec((16, 4 * H), lambda t: (0, 0)),
            pl.BlockSpec((16, 4 * H), lambda t: (0, 0)),
            pl.BlockSpec((2 * B, 4 * H), lambda t: (0, 0)),
            pl.BlockSpec((2 * H, 4 * H), lambda t: (0, 0)),
            pl.BlockSpec((2 * B, 1), lambda t: (0, 0)),
            pl.BlockSpec((2 * B, 1), lambda t: (0, 0)),
        ],
        out_specs=pl.BlockSpec((2 * B, H), lambda t: (0, 0)),
        out_shape=jax.ShapeDtypeStruct((2 * B, H), jnp.float32),
        scratch_shapes=[
            pltpu.VMEM((128, 4 * H), jnp.float32),
            pltpu.VMEM((2 * B, H), jnp.float32),
            pltpu.VMEM((2 * B, H), jnp.float32),
        ],
    )(seq2r, embp, wft, wbt, bfull, whh_cat, lo, hi)


def kernel(seq, seq_len, edge_index, graph_ids, emb, Wih_f, Whh_f, b_f,
           Wih_b, Whh_b, b_b, W1, b1, W2, b2, W3, b3, Wr, br):
    f32 = jnp.float32

    npadd = EPAD - N_EDGES
    dummy = (N_NODES + (jnp.arange(npadd, dtype=jnp.int32) % (NPAD - N_NODES))
             ).astype(jnp.int32)
    src = jnp.concatenate([edge_index[0].astype(jnp.int32), dummy])
    dst = jnp.concatenate([edge_index[1].astype(jnp.int32), dummy])
    srci = src.reshape(16, CHUNKS, 128)
    dsti = dst.reshape(16, CHUNKS, 128)

    deg, s1 = _deg_s1_kernel(srci, dsti)
    degc = deg.reshape(NPAD, 1)

    h1, h1a, h1b = pl.pallas_call(
        _layer1_body,
        out_shape=(jax.ShapeDtypeStruct((NPAD, 128), f32),
                   jax.ShapeDtypeStruct((NPAD, 64), f32),
                   jax.ShapeDtypeStruct((NPAD, 64), f32)),
    )(degc, s1.reshape(NPAD, 1), W1.astype(f32).reshape(1, 128),
      b1.astype(f32).reshape(1, 128))

    s2a, s2b = _agg64(h1a, h1b, srci, dsti)

    rb = pl.BlockSpec((_RB, 64), lambda t: (t, 0))
    rb1 = pl.BlockSpec((_RB, 1), lambda t: (t, 0))
    rb128 = pl.BlockSpec((_RB, 128), lambda t: (t, 0))
    rb256 = pl.BlockSpec((_RB, 256), lambda t: (t, 0))
    h2, h2a, h2b, h2c, h2d = pl.pallas_call(
        _layer2_body,
        grid=(_NRB,),
        in_specs=[rb, rb, rb1, rb128,
                  pl.BlockSpec((128, 256), lambda t: (0, 0)),
                  pl.BlockSpec((1, 256), lambda t: (0, 0))],
        out_specs=(rb256, rb, rb, rb, rb),
        out_shape=(jax.ShapeDtypeStruct((NPAD, 256), f32),
                   jax.ShapeDtypeStruct((NPAD, 64), f32),
                   jax.ShapeDtypeStruct((NPAD, 64), f32),
                   jax.ShapeDtypeStruct((NPAD, 64), f32),
                   jax.ShapeDtypeStruct((NPAD, 64), f32)),
    )(s2a, s2b, degc, h1, W2.astype(f32).T, b2.astype(f32).reshape(1, 256))

    s3a, s3b = _agg64(h2a, h2b, srci, dsti)
    s3c, s3d = _agg64(h2c, h2d, srci, dsti)

    a = jnp.swapaxes(seq, 0, 1).astype(jnp.int32)
    seq2r = jnp.concatenate([a, a[::-1]], axis=1).reshape(T // 2, 2, 2 * B)
    embp = jnp.zeros((32, 16), f32).at[0:21, 0:10].set(emb.astype(f32))
    wft = jnp.zeros((16, 4 * H), f32).at[0:10].set(Wih_f.astype(f32).T)
    wbt = jnp.zeros((16, 4 * H), f32).at[0:10].set(Wih_b.astype(f32).T)
    sl = seq_len.astype(jnp.int32)
    lo = jnp.concatenate([jnp.zeros((B,), jnp.int32), T - sl]).reshape(2 * B, 1)
    hi = jnp.concatenate([sl, jnp.full((B,), T, jnp.int32)]).reshape(2 * B, 1)
    whh_cat = jnp.concatenate(
        [Whh_f.astype(f32).T, Whh_b.astype(f32).T], axis=0)
    bfull = jnp.concatenate(
        [jnp.broadcast_to(b_f.astype(f32), (B, 4 * H)),
         jnp.broadcast_to(b_b.astype(f32), (B, 4 * H))], axis=0)
    hout = _run_lstm(seq2r, embp, wft, wbt, bfull, whh_cat, lo, hi)
    concat_o = jnp.concatenate([hout[:B], hout[B:]], axis=1)

    gidp = jnp.concatenate(
        [graph_ids.astype(jnp.int32),
         jnp.full((NPAD - N_NODES,), 1 << 29, jnp.int32)]).reshape(NPAD, 1)
    out = pl.pallas_call(
        _layer3_body,
        grid=(_NRB,),
        in_specs=[rb, rb, rb, rb, rb1, rb256,
                  pl.BlockSpec((256, 128), lambda t: (0, 0)),
                  pl.BlockSpec((1, 128), lambda t: (0, 0)),
                  pl.BlockSpec((_RB, 1), lambda t: (t, 0)),
                  pl.BlockSpec((B, B), lambda t: (0, 0)),
                  pl.BlockSpec((B, 1), lambda t: (0, 0)),
                  pl.BlockSpec((B, 1), lambda t: (0, 0)),
                  pl.BlockSpec((1, 1), lambda t: (0, 0))],
        out_specs=pl.BlockSpec((B, 1), lambda t: (0, 0)),
        out_shape=jax.ShapeDtypeStruct((B, 1), f32),
        scratch_shapes=[pltpu.VMEM((B, B), f32), pltpu.VMEM((B, 1), f32)],
    )(s3a, s3b, s3c, s3d, degc, h2,
      W3.astype(f32).T, b3.astype(f32).reshape(1, 128), gidp,
      concat_o, Wr.astype(f32)[:, :128].T, Wr.astype(f32)[:, 128:].T,
      br.astype(f32).reshape(1, 1))
    return out
